# Initial kernel scaffold; baseline (speedup 1.0000x reference)
import jax
import jax.numpy as jnp
from jax import lax
from jax.experimental import pallas as pl
from jax.experimental.pallas import tpu as pltpu
from jax.experimental.pallas import tpu_sc as plsc

N0, N1, N2 = 100000, 25000, 2500
E0, E1 = 400000, 40000
F32 = jnp.float32

NTILES = 16
CHUNK = 128

EPT0 = E0 // NTILES
NCH0 = -(-EPT0 // CHUNK)
ACC0 = 25008
STR0 = ACC0 // NTILES

EPT1 = E1 // NTILES
NCH1 = -(-EPT1 // CHUNK)
ACC1 = 2512
STR1 = ACC1 // NTILES


def _zero_rows(ref, nrows, width):
    g = width // 16

    def body(i, _):
        r = i // g
        k = i % g
        ref[r, pl.ds(k * 16, 16)] = jnp.zeros((16,), F32)
        return 0

    lax.fori_loop(0, nrows * g, body, 0)


def _fill_ones(ref, nrows):
    def body(i, _):
        ref[i, pl.ds(0, 16)] = jnp.ones((16,), F32)
        return 0

    lax.fori_loop(0, nrows, body, 0)


def _zero_stripe(shared, zsrc, base, stripe):
    nfull = stripe // CHUNK
    rem = stripe - nfull * CHUNK

    def body(i, _):
        pltpu.sync_copy(zsrc, shared.at[pl.ds(base + i * CHUNK, CHUNK)])
        return 0

    lax.fori_loop(0, nfull, body, 0)
    if rem:
        pltpu.sync_copy(zsrc.at[pl.ds(0, rem)],
                        shared.at[pl.ds(base + nfull * CHUNK, rem)])


def _make_sc_agg(nch, acc_rows, stripe, feat):
    mesh = plsc.VectorSubcoreMesh(core_axis_name="c", subcore_axis_name="s")

    def body(src_hbm, dst_hbm, tab_hbm, sum_hbm, cnt_hbm,
             src_v, dst_v, gbuf, obuf, acc_sh, cnt_sh, sem):
        c = lax.axis_index("c")
        s = lax.axis_index("s")
        base = s * stripe

        _zero_rows(gbuf, CHUNK, feat)
        _zero_rows(obuf, CHUNK, 16)
        _zero_stripe(acc_sh, gbuf, base, stripe)

        @pl.when(c == 0)
        def _():
            _zero_stripe(cnt_sh, obuf, base, stripe)

        _fill_ones(obuf, CHUNK)

        plsc.subcore_barrier()

        pltpu.sync_copy(src_hbm.at[c, s], src_v)
        pltpu.sync_copy(dst_hbm.at[s], dst_v)

        def edge_chunk(j, _):
            pltpu.async_copy(tab_hbm.at[src_v.at[j]], gbuf, sem).wait()
            pltpu.sync_copy(gbuf, acc_sh.at[dst_v.at[j]], add=True)

            @pl.when(c == 0)
            def _():
                pltpu.sync_copy(obuf, cnt_sh.at[dst_v.at[j]], add=True)

            return 0

        lax.fori_loop(0, nch, edge_chunk, 0)

        plsc.subcore_barrier()

        pltpu.sync_copy(acc_sh.at[pl.ds(base, stripe)],
                        sum_hbm.at[c, pl.ds(base, stripe)])

        @pl.when(c == 0)
        def _():
            pltpu.sync_copy(cnt_sh.at[pl.ds(base, stripe)],
                            cnt_hbm.at[pl.ds(base, stripe)])

    return pl.kernel(
        body,
        out_type=(
            jax.ShapeDtypeStruct((2, acc_rows, feat), F32),
            jax.ShapeDtypeStruct((acc_rows, 16), F32),
        ),
        mesh=mesh,
        scratch_types=[
            pltpu.VMEM((nch, CHUNK), jnp.int32),
            pltpu.VMEM((nch, CHUNK), jnp.int32),
            pltpu.VMEM((CHUNK, feat), F32),
            pltpu.VMEM((CHUNK, 16), F32),
            pltpu.VMEM_SHARED((acc_rows, feat), F32),
            pltpu.VMEM_SHARED((acc_rows, 16), F32),
            pltpu.SemaphoreType.DMA,
        ],
    )


_sc_agg0 = _make_sc_agg(NCH0, ACC0, STR0, 64)
_sc_agg1 = _make_sc_agg(NCH1, ACC1, STR1, 128)


def _pad_edges(src, dst, nch, nsrc, dummy):
    ept = src.shape[0] // NTILES
    pert = nch * CHUNK - ept
    srcp = jnp.concatenate(
        [src.reshape(NTILES, ept), jnp.zeros((NTILES, pert), jnp.int32)],
        axis=1).reshape(NTILES, nch, CHUNK)
    dstp = jnp.concatenate(
        [dst.reshape(NTILES, ept), jnp.full((NTILES, pert), dummy, jnp.int32)],
        axis=1).reshape(NTILES, nch, CHUNK)
    src_idx = jnp.stack([srcp, srcp + nsrc])
    return src_idx, dstp


def _k1_body(x_ref, hh_ref, w_ref, b_ref, o_ref):
    h = jnp.dot(x_ref[...], w_ref[...], preferred_element_type=F32)
    h = jnp.maximum(h + b_ref[...], 0.0) - hh_ref[...]
    o_ref[0] = h[:, :64]
    o_ref[1] = h[:, 64:]


def _k1(x, hh, w, b):
    blk = 2000
    grid = N0 // blk
    return pl.pallas_call(
        _k1_body,
        grid=(grid,),
        in_specs=[
            pl.BlockSpec((blk, 128), lambda i: (i, 0)),
            pl.BlockSpec((blk, 128), lambda i: (i, 0)),
            pl.BlockSpec((128, 128), lambda i: (0, 0)),
            pl.BlockSpec((1, 128), lambda i: (0, 0)),
        ],
        out_specs=pl.BlockSpec((2, blk, 64), lambda i: (0, i, 0)),
        out_shape=jax.ShapeDtypeStruct((2, N0, 64), F32),
    )(x, hh, w, b.reshape(1, 128))


def _k3_body(s_ref, c_ref, a_ref, w_ref, b_ref, hh_ref, o_ref):
    ssum = jnp.concatenate([s_ref[0], s_ref[1]], axis=1)
    cnt = c_ref[...][:, :1]
    mean = ssum / jnp.maximum(cnt, 1.0)
    h1 = jnp.dot(mean + a_ref[...], w_ref[...], preferred_element_type=F32)
    h1 = h1 + b_ref[...]
    o_ref[0] = h1 - hh_ref[...][:, :128]
    o_ref[1] = jnp.maximum(h1, 0.0) - hh_ref[...][:, 128:]


def _k3(sum0, cnt0, agg, w, b, hh):
    blk = 1000
    grid = N1 // blk
    return pl.pallas_call(
        _k3_body,
        grid=(grid,),
        in_specs=[
            pl.BlockSpec((2, blk, 64), lambda i: (0, i, 0)),
            pl.BlockSpec((blk, 16), lambda i: (i, 0)),
            pl.BlockSpec((blk, 128), lambda i: (i, 0)),
            pl.BlockSpec((128, 128), lambda i: (0, 0)),
            pl.BlockSpec((1, 128), lambda i: (0, 0)),
            pl.BlockSpec((blk, 256), lambda i: (i, 0)),
        ],
        out_specs=pl.BlockSpec((2, blk, 128), lambda i: (0, i, 0)),
        out_shape=jax.ShapeDtypeStruct((2, N1, 128), F32),
    )(sum0, cnt0, agg, w, b.reshape(1, 128), hh)


def _k5_body(s_ref, c_ref, a_ref, w_ref, b_ref, o_ref):
    ssum = jnp.concatenate([s_ref[0], s_ref[1]], axis=1)[:N2]
    cnt = c_ref[...][:N2, :1]
    h = ssum / jnp.maximum(cnt, 1.0) + a_ref[...]
    o_ref[...] = jnp.dot(h, w_ref[...], preferred_element_type=F32) + b_ref[...]


def _k5(sum1, cnt1, agg, w, b):
    return pl.pallas_call(
        _k5_body,
        out_shape=jax.ShapeDtypeStruct((N2, 64), F32),
    )(sum1, cnt1, agg, w, b.reshape(1, 64))


@jax.jit
def kernel(preprocess, h_hist_0, h_hist_1, agg_h_0, agg_h_1,
           edge_index_0, edge_index_1, W0, b0, W1, b1, W2, b2):
    src_idx0, dst_idx0 = _pad_edges(edge_index_0[0], edge_index_0[1],
                                    NCH0, N0, 25000)
    src_idx1, dst_idx1 = _pad_edges(edge_index_1[0], edge_index_1[1],
                                    NCH1, N1, 2500)

    h_tab = _k1(preprocess, h_hist_0, W0, b0).reshape(2 * N0, 64)
    sum0, cnt0 = _sc_agg0(src_idx0, dst_idx0, h_tab)
    h1_tab = _k3(sum0, cnt0, agg_h_0, W1, b1, h_hist_1).reshape(2 * N1, 128)
    sum1, cnt1 = _sc_agg1(src_idx1, dst_idx1, h1_tab)
    return _k5(sum1, cnt1, agg_h_1, W2, b2)


# speedup vs baseline: 3.8835x; 3.8835x over previous
import jax
import jax.numpy as jnp
from jax import lax
from jax.experimental import pallas as pl
from jax.experimental.pallas import tpu as pltpu
from jax.experimental.pallas import tpu_sc as plsc

N0, N1, N2 = 100000, 25000, 2500
E0, E1 = 400000, 40000
F32 = jnp.float32

NTILES = 16
CHUNK = 128

NCH0, NB0 = 196, 14
ACC0 = 25088
STR0 = ACC0 // NTILES

NCH1, NB1 = 20, 4
ACC1 = 2512
STR1 = ACC1 // NTILES


def _fill_rows(ref, nrows, width, value):
    g = width // 16

    def body(i, _):
        ref[i // g, pl.ds((i % g) * 16, 16)] = jnp.full((16,), value, F32)
        return 0

    lax.fori_loop(0, nrows * g, body, 0)


def _zero_stripe(shared, zsrc, base, stripe):
    nfull = stripe // CHUNK
    rem = stripe - nfull * CHUNK

    def body(i, _):
        pltpu.sync_copy(zsrc, shared.at[pl.ds(base + i * CHUNK, CHUNK)])
        return 0

    if nfull:
        lax.fori_loop(0, nfull, body, 0)
    if rem:
        pltpu.sync_copy(zsrc.at[pl.ds(0, rem)],
                        shared.at[pl.ds(base + nfull * CHUNK, rem)])


def _make_sc_agg(nfpass, nch, nbatch, acc_rows, stripe, sec_rows):
    mesh = plsc.VectorSubcoreMesh(core_axis_name="c", subcore_axis_name="s")
    bch = nch // nbatch

    def body(src_hbm, dst_hbm, tab_hbm, sum_hbm,
             srcb, dstb, gbuf, acc_sh, sem):
        c = lax.axis_index("c")
        s = lax.axis_index("s")
        base = s * stripe

        for p in range(nfpass + 1):
            count_pass = p == nfpass
            _fill_rows(gbuf, CHUNK, 64, 0.0)
            _zero_stripe(acc_sh, gbuf, base, stripe)
            if count_pass:
                _fill_rows(gbuf, CHUNK, 64, 1.0)
            plsc.subcore_barrier()

            off = (2 * p + c) * sec_rows

            if count_pass:
                b_lo = c * (nbatch // 2)
                b_hi = (c + 1) * (nbatch // 2)
            else:
                b_lo, b_hi = 0, nbatch

            def batch_body(b, _):
                pltpu.sync_copy(dst_hbm.at[s, pl.ds(b * bch, bch)], dstb)
                if not count_pass:
                    pltpu.sync_copy(src_hbm.at[s, pl.ds(b * bch, bch)], srcb)

                    def add_off(i, _):
                        sl = (i // 8, pl.ds((i % 8) * 16, 16))
                        srcb[sl] = srcb[sl] + off
                        return 0

                    lax.fori_loop(0, bch * 8, add_off, 0)

                def chunk_body(j, _):
                    if not count_pass:
                        pltpu.async_copy(
                            tab_hbm.at[srcb.at[j]], gbuf, sem).wait()
                    pltpu.sync_copy(gbuf, acc_sh.at[dstb.at[j]],
                                    add=True)
                    return 0

                lax.fori_loop(0, bch, chunk_body, 0)
                return 0

            lax.fori_loop(b_lo, b_hi, batch_body, 0)

            plsc.subcore_barrier()

            pltpu.sync_copy(
                acc_sh.at[pl.ds(base, stripe)],
                sum_hbm.at[2 * p + c, pl.ds(base, stripe)])

            if p < nfpass:
                plsc.subcore_barrier()

    return pl.kernel(
        body,
        out_type=jax.ShapeDtypeStruct((2 * nfpass + 2, acc_rows, 64), F32),
        mesh=mesh,
        compiler_params=pltpu.CompilerParams(
            use_tc_tiling_on_sc=False, needs_layout_passes=False),
        scratch_types=[
            pltpu.VMEM((bch, CHUNK), jnp.int32),
            pltpu.VMEM((bch, CHUNK), jnp.int32),
            pltpu.VMEM((CHUNK, 64), F32),
            pltpu.VMEM_SHARED((acc_rows, 64), F32),
            pltpu.SemaphoreType.DMA,
        ],
    )


_sc_agg0 = _make_sc_agg(1, NCH0, NB0, ACC0, STR0, N0)
_sc_agg1 = _make_sc_agg(2, NCH1, NB1, ACC1, STR1, N1)


def _pad_edges(src, dst, nch, dummy):
    ept = src.shape[0] // NTILES
    pert = nch * CHUNK - ept
    srcp = jnp.concatenate(
        [src.reshape(NTILES, ept), jnp.zeros((NTILES, pert), jnp.int32)],
        axis=1).reshape(NTILES, nch, CHUNK)
    dstp = jnp.concatenate(
        [dst.reshape(NTILES, ept), jnp.full((NTILES, pert), dummy, jnp.int32)],
        axis=1).reshape(NTILES, nch, CHUNK)
    return srcp, dstp


def _k1_body(x_ref, hh_ref, w_ref, b_ref, o_ref):
    h = jnp.dot(x_ref[...], w_ref[...], preferred_element_type=F32)
    h = jnp.maximum(h + b_ref[...], 0.0) - hh_ref[...]
    o_ref[0] = h[:, :64]
    o_ref[1] = h[:, 64:]


def _k1(x, hh, w, b):
    blk = 2000
    return pl.pallas_call(
        _k1_body,
        grid=(N0 // blk,),
        in_specs=[
            pl.BlockSpec((blk, 128), lambda i: (i, 0)),
            pl.BlockSpec((blk, 128), lambda i: (i, 0)),
            pl.BlockSpec((128, 128), lambda i: (0, 0)),
            pl.BlockSpec((1, 128), lambda i: (0, 0)),
        ],
        out_specs=pl.BlockSpec((2, blk, 64), lambda i: (0, i, 0)),
        out_shape=jax.ShapeDtypeStruct((2, N0, 64), F32),
    )(x, hh, w, b.reshape(1, 128))


def _k3_body(s_ref, a_ref, w_ref, b_ref, hh_ref, o_ref):
    ssum = jnp.concatenate([s_ref[0], s_ref[1]], axis=1)
    cnt = s_ref[2][:, :1] + s_ref[3][:, :1]
    mean = ssum / jnp.maximum(cnt, 1.0)
    h1 = jnp.dot(mean + a_ref[...], w_ref[...], preferred_element_type=F32)
    h1 = h1 + b_ref[...]
    r1 = jnp.maximum(h1, 0.0)
    o_ref[0] = h1[:, :64] - hh_ref[...][:, :64]
    o_ref[1] = h1[:, 64:] - hh_ref[...][:, 64:128]
    o_ref[2] = r1[:, :64] - hh_ref[...][:, 128:192]
    o_ref[3] = r1[:, 64:] - hh_ref[...][:, 192:256]


def _k3(sum0, agg, w, b, hh):
    blk = 1000
    return pl.pallas_call(
        _k3_body,
        grid=(N1 // blk,),
        in_specs=[
            pl.BlockSpec((4, blk, 64), lambda i: (0, i, 0)),
            pl.BlockSpec((blk, 128), lambda i: (i, 0)),
            pl.BlockSpec((128, 128), lambda i: (0, 0)),
            pl.BlockSpec((1, 128), lambda i: (0, 0)),
            pl.BlockSpec((blk, 256), lambda i: (i, 0)),
        ],
        out_specs=pl.BlockSpec((4, blk, 64), lambda i: (0, i, 0)),
        out_shape=jax.ShapeDtypeStruct((4, N1, 64), F32),
    )(sum0, agg, w, b.reshape(1, 128), hh)


def _k5_body(s_ref, a_ref, w_ref, b_ref, o_ref):
    ssum = jnp.concatenate(
        [s_ref[0], s_ref[1], s_ref[2], s_ref[3]], axis=1)[:N2]
    cnt = (s_ref[4][:, :1] + s_ref[5][:, :1])[:N2]
    h = ssum / jnp.maximum(cnt, 1.0) + a_ref[...]
    o_ref[...] = jnp.dot(h, w_ref[...], preferred_element_type=F32) + b_ref[...]


def _k5(sum1, agg, w, b):
    return pl.pallas_call(
        _k5_body,
        out_shape=jax.ShapeDtypeStruct((N2, 64), F32),
    )(sum1, agg, w, b.reshape(1, 64))


@jax.jit
def kernel(preprocess, h_hist_0, h_hist_1, agg_h_0, agg_h_1,
           edge_index_0, edge_index_1, W0, b0, W1, b1, W2, b2):
    src_idx0, dst_idx0 = _pad_edges(edge_index_0[0], edge_index_0[1],
                                    NCH0, 25000)
    src_idx1, dst_idx1 = _pad_edges(edge_index_1[0], edge_index_1[1],
                                    NCH1, 2500)

    h_tab = _k1(preprocess, h_hist_0, W0, b0).reshape(2 * N0, 64)
    sum0 = _sc_agg0(src_idx0, dst_idx0, h_tab)
    h1_tab = _k3(sum0, agg_h_0, W1, b1, h_hist_1).reshape(4 * N1, 64)
    sum1 = _sc_agg1(src_idx1, dst_idx1, h1_tab)
    return _k5(sum1, agg_h_1, W2, b2)


# speedup vs baseline: 4.6122x; 1.1876x over previous
import jax
import jax.numpy as jnp
from jax import lax
from jax.experimental import pallas as pl
from jax.experimental.pallas import tpu as pltpu
from jax.experimental.pallas import tpu_sc as plsc

N0, N1, N2 = 100000, 25000, 2500
E0, E1 = 400000, 40000
F32 = jnp.float32

NTILES = 16
CHUNK = 128

NCH0, NB0 = 196, 14
ACC0 = 25088
STR0 = ACC0 // NTILES

NCH1, NB1 = 20, 4
ACC1 = 2512
STR1 = ACC1 // NTILES


def _fill_rows(ref, nrows, width, value):
    g = width // 16

    def body(i, _):
        ref[i // g, pl.ds((i % g) * 16, 16)] = jnp.full((16,), value, F32)
        return 0

    lax.fori_loop(0, nrows * g, body, 0)


def _zero_stripe(shared, zsrc, base, stripe):
    nfull = stripe // CHUNK
    rem = stripe - nfull * CHUNK

    def body(i, _):
        pltpu.sync_copy(zsrc, shared.at[pl.ds(base + i * CHUNK, CHUNK)])
        return 0

    if nfull:
        lax.fori_loop(0, nfull, body, 0)
    if rem:
        pltpu.sync_copy(zsrc.at[pl.ds(0, rem)],
                        shared.at[pl.ds(base + nfull * CHUNK, rem)])


def _make_sc_agg(nfpass, nch, nbatch, acc_rows, stripe, sec_rows):
    mesh = plsc.VectorSubcoreMesh(core_axis_name="c", subcore_axis_name="s")
    bch = nch // nbatch

    def body(src_hbm, dst_hbm, tab_hbm, sum_hbm,
             srcb, dstb, gbuf_a, gbuf_b, acc_sh, gsem_a, gsem_b, ssem):
        c = lax.axis_index("c")
        s = lax.axis_index("s")
        base = s * stripe
        bufs = ((gbuf_a, gsem_a), (gbuf_b, gsem_b))

        for p in range(nfpass + 1):
            count_pass = p == nfpass
            _fill_rows(gbuf_a, CHUNK, 64, 0.0)
            _zero_stripe(acc_sh, gbuf_a, base, stripe)
            if count_pass:
                _fill_rows(gbuf_a, CHUNK, 64, 1.0)
            plsc.subcore_barrier()

            off = (2 * p + c) * sec_rows

            if count_pass:
                b_lo = c * (nbatch // 2)
                b_hi = (c + 1) * (nbatch // 2)
            else:
                b_lo, b_hi = 0, nbatch

            def batch_body(b, _):
                pltpu.sync_copy(dst_hbm.at[s, pl.ds(b * bch, bch)], dstb)
                if count_pass:
                    hs = [
                        pltpu.async_copy(
                            gbuf_a, acc_sh.at[dstb.at[k]], ssem, add=True)
                        for k in range(bch)
                    ]
                    for h in hs:
                        h.wait()
                    return 0

                pltpu.sync_copy(src_hbm.at[s, pl.ds(b * bch, bch)], srcb)

                def add_off(i, _):
                    sl = (i // 8, pl.ds((i % 8) * 16, 16))
                    srcb[sl] = srcb[sl] + off
                    return 0

                lax.fori_loop(0, bch * 8, add_off, 0)

                g = [None, None]
                g[0] = pltpu.async_copy(
                    tab_hbm.at[srcb.at[0]], gbuf_a, gsem_a)
                if bch > 1:
                    g[1] = pltpu.async_copy(
                        tab_hbm.at[srcb.at[1]], gbuf_b, gsem_b)
                for k in range(bch):
                    buf, gsem = bufs[k % 2]
                    g[k % 2].wait()
                    pltpu.async_copy(
                        buf, acc_sh.at[dstb.at[k]], ssem, add=True).wait()
                    if k + 2 < bch:
                        g[k % 2] = pltpu.async_copy(
                            tab_hbm.at[srcb.at[k + 2]], buf, gsem)
                return 0

            lax.fori_loop(b_lo, b_hi, batch_body, 0)

            plsc.subcore_barrier()

            pltpu.sync_copy(
                acc_sh.at[pl.ds(base, stripe)],
                sum_hbm.at[2 * p + c, pl.ds(base, stripe)])

            if p < nfpass:
                plsc.subcore_barrier()

    return pl.kernel(
        body,
        out_type=jax.ShapeDtypeStruct((2 * nfpass + 2, acc_rows, 64), F32),
        mesh=mesh,
        compiler_params=pltpu.CompilerParams(
            use_tc_tiling_on_sc=False, needs_layout_passes=False),
        scratch_types=[
            pltpu.VMEM((bch, CHUNK), jnp.int32),
            pltpu.VMEM((bch, CHUNK), jnp.int32),
            pltpu.VMEM((CHUNK, 64), F32),
            pltpu.VMEM((CHUNK, 64), F32),
            pltpu.VMEM_SHARED((acc_rows, 64), F32),
            pltpu.SemaphoreType.DMA,
            pltpu.SemaphoreType.DMA,
            pltpu.SemaphoreType.DMA,
        ],
    )


_sc_agg0 = _make_sc_agg(1, NCH0, NB0, ACC0, STR0, N0)
_sc_agg1 = _make_sc_agg(2, NCH1, NB1, ACC1, STR1, N1)


def _pad_edges(src, dst, nch, dummy):
    ept = src.shape[0] // NTILES
    pert = nch * CHUNK - ept
    srcp = jnp.concatenate(
        [src.reshape(NTILES, ept), jnp.zeros((NTILES, pert), jnp.int32)],
        axis=1).reshape(NTILES, nch, CHUNK)
    dstp = jnp.concatenate(
        [dst.reshape(NTILES, ept), jnp.full((NTILES, pert), dummy, jnp.int32)],
        axis=1).reshape(NTILES, nch, CHUNK)
    return srcp, dstp


def _k1_body(x_ref, hh_ref, w_ref, b_ref, o_ref):
    h = jnp.dot(x_ref[...], w_ref[...], preferred_element_type=F32)
    h = jnp.maximum(h + b_ref[...], 0.0) - hh_ref[...]
    o_ref[0] = h[:, :64]
    o_ref[1] = h[:, 64:]


def _k1(x, hh, w, b):
    blk = 2000
    return pl.pallas_call(
        _k1_body,
        grid=(N0 // blk,),
        in_specs=[
            pl.BlockSpec((blk, 128), lambda i: (i, 0)),
            pl.BlockSpec((blk, 128), lambda i: (i, 0)),
            pl.BlockSpec((128, 128), lambda i: (0, 0)),
            pl.BlockSpec((1, 128), lambda i: (0, 0)),
        ],
        out_specs=pl.BlockSpec((2, blk, 64), lambda i: (0, i, 0)),
        out_shape=jax.ShapeDtypeStruct((2, N0, 64), F32),
    )(x, hh, w, b.reshape(1, 128))


def _k3_body(s_ref, a_ref, w_ref, b_ref, hh_ref, o_ref):
    ssum = jnp.concatenate([s_ref[0], s_ref[1]], axis=1)
    cnt = s_ref[2][:, :1] + s_ref[3][:, :1]
    mean = ssum / jnp.maximum(cnt, 1.0)
    h1 = jnp.dot(mean + a_ref[...], w_ref[...], preferred_element_type=F32)
    h1 = h1 + b_ref[...]
    r1 = jnp.maximum(h1, 0.0)
    o_ref[0] = h1[:, :64] - hh_ref[...][:, :64]
    o_ref[1] = h1[:, 64:] - hh_ref[...][:, 64:128]
    o_ref[2] = r1[:, :64] - hh_ref[...][:, 128:192]
    o_ref[3] = r1[:, 64:] - hh_ref[...][:, 192:256]


def _k3(sum0, agg, w, b, hh):
    blk = 1000
    return pl.pallas_call(
        _k3_body,
        grid=(N1 // blk,),
        in_specs=[
            pl.BlockSpec((4, blk, 64), lambda i: (0, i, 0)),
            pl.BlockSpec((blk, 128), lambda i: (i, 0)),
            pl.BlockSpec((128, 128), lambda i: (0, 0)),
            pl.BlockSpec((1, 128), lambda i: (0, 0)),
            pl.BlockSpec((blk, 256), lambda i: (i, 0)),
        ],
        out_specs=pl.BlockSpec((4, blk, 64), lambda i: (0, i, 0)),
        out_shape=jax.ShapeDtypeStruct((4, N1, 64), F32),
    )(sum0, agg, w, b.reshape(1, 128), hh)


def _k5_body(s_ref, a_ref, w_ref, b_ref, o_ref):
    ssum = jnp.concatenate(
        [s_ref[0], s_ref[1], s_ref[2], s_ref[3]], axis=1)[:N2]
    cnt = (s_ref[4][:, :1] + s_ref[5][:, :1])[:N2]
    h = ssum / jnp.maximum(cnt, 1.0) + a_ref[...]
    o_ref[...] = jnp.dot(h, w_ref[...], preferred_element_type=F32) + b_ref[...]


def _k5(sum1, agg, w, b):
    return pl.pallas_call(
        _k5_body,
        out_shape=jax.ShapeDtypeStruct((N2, 64), F32),
    )(sum1, agg, w, b.reshape(1, 64))


@jax.jit
def kernel(preprocess, h_hist_0, h_hist_1, agg_h_0, agg_h_1,
           edge_index_0, edge_index_1, W0, b0, W1, b1, W2, b2):
    src_idx0, dst_idx0 = _pad_edges(edge_index_0[0], edge_index_0[1],
                                    NCH0, 25000)
    src_idx1, dst_idx1 = _pad_edges(edge_index_1[0], edge_index_1[1],
                                    NCH1, 2500)

    h_tab = _k1(preprocess, h_hist_0, W0, b0).reshape(2 * N0, 64)
    sum0 = _sc_agg0(src_idx0, dst_idx0, h_tab)
    h1_tab = _k3(sum0, agg_h_0, W1, b1, h_hist_1).reshape(4 * N1, 64)
    sum1 = _sc_agg1(src_idx1, dst_idx1, h1_tab)
    return _k5(sum1, agg_h_1, W2, b2)


# speedup vs baseline: 4.6683x; 1.0122x over previous
import jax
import jax.numpy as jnp
from jax import lax
from jax.experimental import pallas as pl
from jax.experimental.pallas import tpu as pltpu
from jax.experimental.pallas import tpu_sc as plsc

N0, N1, N2 = 100000, 25000, 2500
E0, E1 = 400000, 40000
F32 = jnp.float32

NTILES = 16
CHUNK = 128

NCH0, NB0 = 196, 7
ACC0 = 25088
STR0 = ACC0 // NTILES

NCH1, NB1 = 20, 4
ACC1 = 2512
STR1 = ACC1 // NTILES


def _fill_rows(ref, nrows, width, value):
    g = width // 16

    def body(i, _):
        ref[i // g, pl.ds((i % g) * 16, 16)] = jnp.full((16,), value, F32)
        return 0

    lax.fori_loop(0, nrows * g, body, 0)


def _zero_stripe(shared, zsrc, base, stripe):
    nfull = stripe // CHUNK
    rem = stripe - nfull * CHUNK

    def body(i, _):
        pltpu.sync_copy(zsrc, shared.at[pl.ds(base + i * CHUNK, CHUNK)])
        return 0

    if nfull:
        lax.fori_loop(0, nfull, body, 0)
    if rem:
        pltpu.sync_copy(zsrc.at[pl.ds(0, rem)],
                        shared.at[pl.ds(base + nfull * CHUNK, rem)])


def _make_sc_agg(nfpass, nch, nbatch, acc_rows, stripe, sec_rows):
    mesh = plsc.VectorSubcoreMesh(core_axis_name="c", subcore_axis_name="s")
    bch = nch // nbatch

    def body(src_hbm, dst_hbm, tab_hbm, sum_hbm,
             srcb, dstb, gbuf_a, gbuf_b, acc_sh, gsem_a, gsem_b, ssem):
        c = lax.axis_index("c")
        s = lax.axis_index("s")
        base = s * stripe
        bufs = ((gbuf_a, gsem_a), (gbuf_b, gsem_b))

        for p in range(nfpass + 1):
            count_pass = p == nfpass
            _fill_rows(gbuf_a, CHUNK, 64, 0.0)
            _zero_stripe(acc_sh, gbuf_a, base, stripe)
            if count_pass:
                _fill_rows(gbuf_a, CHUNK, 64, 1.0)
            plsc.subcore_barrier()

            off = (2 * p + c) * sec_rows

            if count_pass:
                half = nbatch // 2
                b_lo = c * half
                b_hi = half + c * (nbatch - half)
            else:
                b_lo, b_hi = 0, nbatch

            def batch_body(b, _):
                pltpu.sync_copy(dst_hbm.at[s, pl.ds(b * bch, bch)], dstb)
                if count_pass:
                    hs = [
                        pltpu.async_copy(
                            gbuf_a, acc_sh.at[dstb.at[k]], ssem, add=True)
                        for k in range(bch)
                    ]
                    for h in hs:
                        h.wait()
                    return 0

                pltpu.sync_copy(src_hbm.at[s, pl.ds(b * bch, bch)], srcb)

                def add_off(i, _):
                    sl = (i // 8, pl.ds((i % 8) * 16, 16))
                    srcb[sl] = srcb[sl] + off
                    return 0

                lax.fori_loop(0, bch * 8, add_off, 0)

                g = [None, None]
                g[0] = pltpu.async_copy(
                    tab_hbm.at[srcb.at[0]], gbuf_a, gsem_a)
                if bch > 1:
                    g[1] = pltpu.async_copy(
                        tab_hbm.at[srcb.at[1]], gbuf_b, gsem_b)
                for k in range(bch):
                    buf, gsem = bufs[k % 2]
                    g[k % 2].wait()
                    pltpu.async_copy(
                        buf, acc_sh.at[dstb.at[k]], ssem, add=True).wait()
                    if k + 2 < bch:
                        g[k % 2] = pltpu.async_copy(
                            tab_hbm.at[srcb.at[k + 2]], buf, gsem)
                return 0

            lax.fori_loop(b_lo, b_hi, batch_body, 0)

            plsc.subcore_barrier()

            pltpu.sync_copy(
                acc_sh.at[pl.ds(base, stripe)],
                sum_hbm.at[2 * p + c, pl.ds(base, stripe)])

            if p < nfpass:
                plsc.subcore_barrier()

    return pl.kernel(
        body,
        out_type=jax.ShapeDtypeStruct((2 * nfpass + 2, acc_rows, 64), F32),
        mesh=mesh,
        compiler_params=pltpu.CompilerParams(
            use_tc_tiling_on_sc=False, needs_layout_passes=False),
        scratch_types=[
            pltpu.VMEM((bch, CHUNK), jnp.int32),
            pltpu.VMEM((bch, CHUNK), jnp.int32),
            pltpu.VMEM((CHUNK, 64), F32),
            pltpu.VMEM((CHUNK, 64), F32),
            pltpu.VMEM_SHARED((acc_rows, 64), F32),
            pltpu.SemaphoreType.DMA,
            pltpu.SemaphoreType.DMA,
            pltpu.SemaphoreType.DMA,
        ],
    )


_sc_agg0 = _make_sc_agg(1, NCH0, NB0, ACC0, STR0, N0)
_sc_agg1 = _make_sc_agg(2, NCH1, NB1, ACC1, STR1, N1)


def _pad_edges(src, dst, nch, dummy):
    ept = src.shape[0] // NTILES
    pert = nch * CHUNK - ept
    srcp = jnp.concatenate(
        [src.reshape(NTILES, ept), jnp.zeros((NTILES, pert), jnp.int32)],
        axis=1).reshape(NTILES, nch, CHUNK)
    dstp = jnp.concatenate(
        [dst.reshape(NTILES, ept), jnp.full((NTILES, pert), dummy, jnp.int32)],
        axis=1).reshape(NTILES, nch, CHUNK)
    return srcp, dstp


def _k1_body(x_ref, hh_ref, w_ref, b_ref, o_ref):
    h = jnp.dot(x_ref[...], w_ref[...], preferred_element_type=F32)
    h = jnp.maximum(h + b_ref[...], 0.0) - hh_ref[...]
    o_ref[0] = h[:, :64]
    o_ref[1] = h[:, 64:]


def _k1(x, hh, w, b):
    blk = 2000
    return pl.pallas_call(
        _k1_body,
        grid=(N0 // blk,),
        in_specs=[
            pl.BlockSpec((blk, 128), lambda i: (i, 0)),
            pl.BlockSpec((blk, 128), lambda i: (i, 0)),
            pl.BlockSpec((128, 128), lambda i: (0, 0)),
            pl.BlockSpec((1, 128), lambda i: (0, 0)),
        ],
        out_specs=pl.BlockSpec((2, blk, 64), lambda i: (0, i, 0)),
        out_shape=jax.ShapeDtypeStruct((2, N0, 64), F32),
    )(x, hh, w, b.reshape(1, 128))


def _k3_body(s_ref, a_ref, w_ref, b_ref, hh_ref, o_ref):
    ssum = jnp.concatenate([s_ref[0], s_ref[1]], axis=1)
    cnt = s_ref[2][:, :1] + s_ref[3][:, :1]
    mean = ssum / jnp.maximum(cnt, 1.0)
    h1 = jnp.dot(mean + a_ref[...], w_ref[...], preferred_element_type=F32)
    h1 = h1 + b_ref[...]
    r1 = jnp.maximum(h1, 0.0)
    o_ref[0] = h1[:, :64] - hh_ref[...][:, :64]
    o_ref[1] = h1[:, 64:] - hh_ref[...][:, 64:128]
    o_ref[2] = r1[:, :64] - hh_ref[...][:, 128:192]
    o_ref[3] = r1[:, 64:] - hh_ref[...][:, 192:256]


def _k3(sum0, agg, w, b, hh):
    blk = 1000
    return pl.pallas_call(
        _k3_body,
        grid=(N1 // blk,),
        in_specs=[
            pl.BlockSpec((4, blk, 64), lambda i: (0, i, 0)),
            pl.BlockSpec((blk, 128), lambda i: (i, 0)),
            pl.BlockSpec((128, 128), lambda i: (0, 0)),
            pl.BlockSpec((1, 128), lambda i: (0, 0)),
            pl.BlockSpec((blk, 256), lambda i: (i, 0)),
        ],
        out_specs=pl.BlockSpec((4, blk, 64), lambda i: (0, i, 0)),
        out_shape=jax.ShapeDtypeStruct((4, N1, 64), F32),
    )(sum0, agg, w, b.reshape(1, 128), hh)


def _k5_body(s_ref, a_ref, w_ref, b_ref, o_ref):
    ssum = jnp.concatenate(
        [s_ref[0], s_ref[1], s_ref[2], s_ref[3]], axis=1)[:N2]
    cnt = (s_ref[4][:, :1] + s_ref[5][:, :1])[:N2]
    h = ssum / jnp.maximum(cnt, 1.0) + a_ref[...]
    o_ref[...] = jnp.dot(h, w_ref[...], preferred_element_type=F32) + b_ref[...]


def _k5(sum1, agg, w, b):
    return pl.pallas_call(
        _k5_body,
        out_shape=jax.ShapeDtypeStruct((N2, 64), F32),
    )(sum1, agg, w, b.reshape(1, 64))


@jax.jit
def kernel(preprocess, h_hist_0, h_hist_1, agg_h_0, agg_h_1,
           edge_index_0, edge_index_1, W0, b0, W1, b1, W2, b2):
    src_idx0, dst_idx0 = _pad_edges(edge_index_0[0], edge_index_0[1],
                                    NCH0, 25000)
    src_idx1, dst_idx1 = _pad_edges(edge_index_1[0], edge_index_1[1],
                                    NCH1, 2500)

    h_tab = _k1(preprocess, h_hist_0, W0, b0).reshape(2 * N0, 64)
    sum0 = _sc_agg0(src_idx0, dst_idx0, h_tab)
    h1_tab = _k3(sum0, agg_h_0, W1, b1, h_hist_1).reshape(4 * N1, 64)
    sum1 = _sc_agg1(src_idx1, dst_idx1, h1_tab)
    return _k5(sum1, agg_h_1, W2, b2)
